# Initial kernel scaffold; baseline (speedup 1.0000x reference)
import jax
import jax.numpy as jnp
from jax.experimental import pallas as pl


def kernel(betas, t_int):
    raise NotImplementedError("write your pallas kernel here")


# speedup vs baseline: 4.5755x; 4.5755x over previous
import functools

import jax
import jax.numpy as jnp
from jax import lax
from jax.experimental import pallas as pl
from jax.experimental.pallas import tpu as pltpu
from jax.experimental.pallas import tpu_sc as plsc

_B = 16384
_L = 16


@functools.lru_cache(maxsize=None)
def _build(table_len: int):
    info = plsc.get_sparse_core_info()
    nc, ns = info.num_cores, info.num_subcores
    nw = nc * ns
    b_per_w = _B // nw

    mesh = plsc.VectorSubcoreMesh(core_axis_name="c", subcore_axis_name="s")

    @functools.partial(
        pl.kernel,
        mesh=mesh,
        out_type=jax.ShapeDtypeStruct((_B,), jnp.float32),
        compiler_params=pltpu.CompilerParams(needs_layout_passes=False),
        scratch_types=[
            pltpu.VMEM((table_len,), jnp.float32),
            pltpu.VMEM((b_per_w,), jnp.int32),
            pltpu.VMEM((b_per_w,), jnp.float32),
        ],
    )
    def k(table_hbm, idx_hbm, out_hbm, table_v, idx_v, vals_v):
        wid = lax.axis_index("s") * nc + lax.axis_index("c")
        base = wid * b_per_w
        pltpu.sync_copy(table_hbm, table_v)
        pltpu.sync_copy(idx_hbm.at[pl.ds(base, b_per_w)], idx_v)
        for i in range(b_per_w // _L):
            idx16 = idx_v[pl.ds(i * _L, _L)]
            vals_v[pl.ds(i * _L, _L)] = plsc.load_gather(table_v, [idx16])
        pltpu.sync_copy(vals_v, out_hbm.at[pl.ds(base, b_per_w)])

    return k


def kernel(betas, t_int):
    n = betas.shape[0]
    pad = (-n) % _L
    table = jnp.pad(betas.astype(jnp.float32), (0, pad))
    return _build(n + pad)(table, t_int.astype(jnp.int32))


# speedup vs baseline: 4.6778x; 1.0223x over previous
import functools

import jax
import jax.numpy as jnp
from jax import lax
from jax.experimental import pallas as pl
from jax.experimental.pallas import tpu as pltpu
from jax.experimental.pallas import tpu_sc as plsc

_B = 16384
_L = 16


@functools.lru_cache(maxsize=None)
def _build(table_len: int):
    info = plsc.get_sparse_core_info()
    nc, ns = info.num_cores, info.num_subcores
    nw = nc * ns
    b_per_w = _B // nw

    mesh = plsc.VectorSubcoreMesh(core_axis_name="c", subcore_axis_name="s")

    @functools.partial(
        pl.kernel,
        mesh=mesh,
        out_type=jax.ShapeDtypeStruct((_B,), jnp.float32),
        compiler_params=pltpu.CompilerParams(needs_layout_passes=False),
        scratch_types=[
            pltpu.VMEM((table_len,), jnp.float32),
            pltpu.VMEM((b_per_w,), jnp.int32),
            pltpu.VMEM((b_per_w,), jnp.float32),
            pltpu.SemaphoreType.DMA,
            pltpu.SemaphoreType.DMA,
        ],
    )
    def k(table_hbm, idx_hbm, out_hbm, table_v, idx_v, vals_v, sem_t, sem_i):
        wid = lax.axis_index("s") * nc + lax.axis_index("c")
        base = wid * b_per_w
        cp_t = pltpu.async_copy(table_hbm, table_v, sem_t)
        cp_i = pltpu.async_copy(idx_hbm.at[pl.ds(base, b_per_w)], idx_v, sem_i)
        cp_i.wait()
        cp_t.wait()
        for i in range(b_per_w // _L):
            idx16 = idx_v[pl.ds(i * _L, _L)]
            vals_v[pl.ds(i * _L, _L)] = plsc.load_gather(table_v, [idx16])
        pltpu.sync_copy(vals_v, out_hbm.at[pl.ds(base, b_per_w)])

    return k


def kernel(betas, t_int):
    return _build(betas.shape[0])(betas.astype(jnp.float32),
                                  t_int.astype(jnp.int32))
